# baseline (speedup 1.0000x reference)
import functools

import jax
import jax.numpy as jnp
import numpy as np
from jax import lax
from jax.experimental import pallas as pl
from jax.experimental.pallas import tpu as pltpu
from jax.experimental.pallas import tpu_sc as plsc

_PAD = 64
_ROWS = 128
_LEN = 4096
_WIN = _LEN + 8
_GUARD = 80
_NQ = _LEN // 16
_EDGE = _PAD // 16


def _threefry2x32(k1, k2, x1, x2):
    u32 = np.uint32
    x = [np.asarray(x1, u32).copy(), np.asarray(x2, u32).copy()]
    ks = [u32(k1), u32(k2), u32(k1) ^ u32(k2) ^ u32(0x1BD11BDA)]

    def rl(v, d):
        return ((v << u32(d)) | (v >> u32(32 - d))).astype(u32)

    x[0] = (x[0] + ks[0]).astype(u32)
    x[1] = (x[1] + ks[1]).astype(u32)
    ks_rot = [ks[1], ks[2], ks[0]]
    rots = [(13, 15, 26, 6), (17, 29, 16, 24)]
    for i in range(5):
        for r in rots[0]:
            x[0] = (x[0] + x[1]).astype(u32)
            x[1] = x[0] ^ rl(x[1], r)
        x = [(x[0] + ks_rot[0]).astype(u32),
             (x[1] + ks_rot[1] + u32(i + 1)).astype(u32)]
        ks_rot = ks_rot[1:] + ks_rot[:1]
        rots = rots[1:] + rots[:1]
    return x[0], x[1]


@functools.cache
def _crop_starts(n):
    a, b = _threefry2x32(0, 0, np.zeros(1, np.uint32), np.ones(1, np.uint32))
    rng = np.concatenate([a, b])
    s1, s2 = _threefry2x32(rng[0], rng[1], np.zeros(2, np.uint32),
                           np.arange(2, dtype=np.uint32))
    keys = np.stack([s1, s2], axis=1)
    zero, cnt = np.zeros(128, np.uint32), np.arange(128, dtype=np.uint32)
    h1, h2 = _threefry2x32(keys[0, 0], keys[0, 1], zero, cnt)
    l1, l2 = _threefry2x32(keys[1, 0], keys[1, 1], zero, cnt)
    hi, lo = h1 ^ h2, l1 ^ l2
    minval = -64
    span = np.uint64(n - 4096 + 64 - minval)
    mult = np.uint64(2**16) % span
    mult = (mult * mult) % np.uint64(2**32) % span
    off = (hi.astype(np.uint64) % span) * mult + lo.astype(np.uint64) % span
    off = off % np.uint64(2**32) % span
    starts = (minval + off.astype(np.int64)).astype(np.int32)
    return np.clip(starts, -_PAD, n + _PAD - _LEN)


def _crops_sc(data, starts):
    n = data.shape[0]
    assert n % 8 == 0 and (n - _WIN) % 8 == 0
    mesh = plsc.VectorSubcoreMesh(core_axis_name="c", subcore_axis_name="s")

    @functools.partial(
        pl.kernel,
        out_type=jax.ShapeDtypeStruct((_ROWS * _LEN,), jnp.float32),
        mesh=mesh,
        scratch_types=[
            pltpu.VMEM((32,), jnp.int32),
            pltpu.VMEM((_PAD + _WIN + _GUARD,), jnp.float32),
            pltpu.VMEM((_PAD + _WIN + _GUARD,), jnp.float32),
            pltpu.VMEM((_PAD + _WIN + _GUARD,), jnp.float32),
            pltpu.VMEM((_PAD + _WIN + _GUARD,), jnp.float32),
            pltpu.VMEM((_LEN,), jnp.float32),
            pltpu.VMEM((_LEN,), jnp.float32),
            pltpu.VMEM((_LEN,), jnp.float32),
            pltpu.VMEM((_LEN,), jnp.float32),
            pltpu.SemaphoreType.DMA,
            pltpu.SemaphoreType.DMA,
            pltpu.SemaphoreType.DMA,
            pltpu.SemaphoreType.DMA,
            pltpu.SemaphoreType.DMA,
        ],
    )
    def k(data_hbm, starts_hbm, out_hbm, starts_v,
          win0, win1, win2, win3, row0, row1, row2, row3,
          si0, si1, si2, si3, so):
        wid = lax.axis_index("s") * 2 + lax.axis_index("c")
        wins, rows_v = (win0, win1, win2, win3), (row0, row1, row2, row3)
        s_in = (si0, si1, si2, si3)
        pltpu.sync_copy(starts_hbm.at[pl.ds((wid // 4) * 16, 16)], starts_v.at[pl.ds(0, 16)])
        ss, cs, shifts = [], [], []
        for r in range(4):
            s = starts_v[pl.ds((wid % 4) * 4 + r, 16)][0]
            c = jnp.clip(s - lax.rem(s, 8), 0, n - _WIN)
            ss.append(s)
            cs.append(pl.multiple_of(c, 8))
            shifts.append(_PAD + s - c)

        jvec = lax.broadcasted_iota(jnp.int32, (16,), 0)
        cin = [
            pltpu.async_copy(
                data_hbm.at[pl.ds(cs[r], _WIN)], wins[r].at[pl.ds(_PAD, _WIN)],
                s_in[r])
            for r in range(4)
        ]
        cout = [None] * 4
        for r in range(4):
            cin[r].wait()
            s, sh = ss[r], shifts[r]
            win, row_v = wins[r], rows_v[r]

            @plsc.parallel_loop(0, _NQ, unroll=32)
            def _realign(q):
                row_v[pl.ds(q * 16, 16)] = win[pl.ds(sh + q * 16, 16)]

            @pl.when((s < 0) | (s > n - _LEN))
            def _mask_edges():
                for q in list(range(_EDGE)) + list(range(_NQ - _EDGE, _NQ)):
                    j = jvec + (q * 16)
                    src = j + s
                    val = win[pl.ds(sh + q * 16, 16)]
                    valid = (src >= 0) & (src < n)
                    row_v[pl.ds(q * 16, 16)] = jnp.where(valid, val, 0.0)

            row = wid * 4 + r
            cout[r] = pltpu.async_copy(
                row_v, out_hbm.at[pl.ds(pl.multiple_of(row * _LEN, 8), _LEN)], so)
        for r in range(4):
            cout[r].wait()

    return k(data, starts)


def kernel(data, batch_size, length, p):
    del batch_size, length, p
    n = data.shape[0]
    starts = jnp.asarray(_crop_starts(n), dtype=jnp.int32)
    return _crops_sc(data, starts).reshape(_ROWS, _LEN, 1)


# speedup vs baseline: 1.0231x; 1.0231x over previous
import functools

import jax
import jax.numpy as jnp
import numpy as np
from jax import lax
from jax.experimental import pallas as pl
from jax.experimental.pallas import tpu as pltpu
from jax.experimental.pallas import tpu_sc as plsc

_PAD = 64
_ROWS = 128
_LEN = 4096
_WIN = _LEN + 8
_GUARD = 80
_NQ = _LEN // 16
_EDGE = _PAD // 16


def _threefry2x32(k1, k2, x1, x2):
    u32 = np.uint32
    x = [np.asarray(x1, u32).copy(), np.asarray(x2, u32).copy()]
    ks = [u32(k1), u32(k2), u32(k1) ^ u32(k2) ^ u32(0x1BD11BDA)]

    def rl(v, d):
        return ((v << u32(d)) | (v >> u32(32 - d))).astype(u32)

    x[0] = (x[0] + ks[0]).astype(u32)
    x[1] = (x[1] + ks[1]).astype(u32)
    ks_rot = [ks[1], ks[2], ks[0]]
    rots = [(13, 15, 26, 6), (17, 29, 16, 24)]
    for i in range(5):
        for r in rots[0]:
            x[0] = (x[0] + x[1]).astype(u32)
            x[1] = x[0] ^ rl(x[1], r)
        x = [(x[0] + ks_rot[0]).astype(u32),
             (x[1] + ks_rot[1] + u32(i + 1)).astype(u32)]
        ks_rot = ks_rot[1:] + ks_rot[:1]
        rots = rots[1:] + rots[:1]
    return x[0], x[1]


@functools.cache
def _crop_starts(n):
    a, b = _threefry2x32(0, 0, np.zeros(1, np.uint32), np.ones(1, np.uint32))
    rng = np.concatenate([a, b])
    s1, s2 = _threefry2x32(rng[0], rng[1], np.zeros(2, np.uint32),
                           np.arange(2, dtype=np.uint32))
    keys = np.stack([s1, s2], axis=1)
    zero, cnt = np.zeros(128, np.uint32), np.arange(128, dtype=np.uint32)
    h1, h2 = _threefry2x32(keys[0, 0], keys[0, 1], zero, cnt)
    l1, l2 = _threefry2x32(keys[1, 0], keys[1, 1], zero, cnt)
    hi, lo = h1 ^ h2, l1 ^ l2
    minval = -64
    span = np.uint64(n - 4096 + 64 - minval)
    mult = np.uint64(2**16) % span
    mult = (mult * mult) % np.uint64(2**32) % span
    off = (hi.astype(np.uint64) % span) * mult + lo.astype(np.uint64) % span
    off = off % np.uint64(2**32) % span
    starts = (minval + off.astype(np.int64)).astype(np.int32)
    return np.clip(starts, -_PAD, n + _PAD - _LEN)


def _crops_sc(data, starts):
    n = data.shape[0]
    assert n % 8 == 0 and (n - _WIN) % 8 == 0
    mesh = plsc.VectorSubcoreMesh(core_axis_name="c", subcore_axis_name="s")

    @functools.partial(
        pl.kernel,
        out_type=jax.ShapeDtypeStruct((_ROWS * _LEN,), jnp.float32),
        mesh=mesh,
        scratch_types=[
            pltpu.VMEM((32,), jnp.int32),
            pltpu.VMEM((_PAD + _WIN + _GUARD,), jnp.float32),
            pltpu.VMEM((_PAD + _WIN + _GUARD,), jnp.float32),
            pltpu.VMEM((_PAD + _WIN + _GUARD,), jnp.float32),
            pltpu.VMEM((_PAD + _WIN + _GUARD,), jnp.float32),
            pltpu.VMEM((_LEN,), jnp.float32),
            pltpu.VMEM((_LEN,), jnp.float32),
            pltpu.VMEM((_LEN,), jnp.float32),
            pltpu.VMEM((_LEN,), jnp.float32),
            pltpu.SemaphoreType.DMA,
            pltpu.SemaphoreType.DMA,
            pltpu.SemaphoreType.DMA,
            pltpu.SemaphoreType.DMA,
            pltpu.SemaphoreType.DMA,
        ],
    )
    def k(data_hbm, starts_hbm, out_hbm, starts_v,
          win0, win1, win2, win3, row0, row1, row2, row3,
          si0, si1, si2, si3, so):
        wid = lax.axis_index("s") * 2 + lax.axis_index("c")
        wins, rows_v = (win0, win1, win2, win3), (row0, row1, row2, row3)
        s_in = (si0, si1, si2, si3)
        pltpu.sync_copy(starts_hbm.at[pl.ds((wid // 4) * 16, 16)], starts_v.at[pl.ds(0, 16)])
        ss, cs, shifts = [], [], []
        for r in range(4):
            s = starts_v[pl.ds((wid % 4) * 4 + r, 16)][0]
            c = jnp.clip(s - lax.rem(s, 8), 0, n - _WIN)
            ss.append(s)
            cs.append(pl.multiple_of(c, 8))
            shifts.append(_PAD + s - c)

        jvec = lax.broadcasted_iota(jnp.int32, (16,), 0)
        cin = [
            pltpu.async_copy(
                data_hbm.at[pl.ds(cs[r], _WIN)], wins[r].at[pl.ds(_PAD, _WIN)],
                s_in[r])
            for r in range(4)
        ]
        cout = [None] * 4
        for ra in (0, 2):
            rb = ra + 1
            cin[ra].wait()
            cin[rb].wait()
            win_a, row_a, sh_a = wins[ra], rows_v[ra], shifts[ra]
            win_b, row_b, sh_b = wins[rb], rows_v[rb], shifts[rb]

            @plsc.parallel_loop(0, _NQ, unroll=16)
            def _realign(q):
                row_a[pl.ds(q * 16, 16)] = win_a[pl.ds(sh_a + q * 16, 16)]
                row_b[pl.ds(q * 16, 16)] = win_b[pl.ds(sh_b + q * 16, 16)]

            for r in (ra, rb):
                s, sh = ss[r], shifts[r]
                win, row_v = wins[r], rows_v[r]

                @pl.when((s < 0) | (s > n - _LEN))
                def _mask_edges():
                    for q in list(range(_EDGE)) + list(range(_NQ - _EDGE, _NQ)):
                        j = jvec + (q * 16)
                        src = j + s
                        val = win[pl.ds(sh + q * 16, 16)]
                        valid = (src >= 0) & (src < n)
                        row_v[pl.ds(q * 16, 16)] = jnp.where(valid, val, 0.0)

                row = wid * 4 + r
                cout[r] = pltpu.async_copy(
                    row_v, out_hbm.at[pl.ds(pl.multiple_of(row * _LEN, 8), _LEN)], so)
        for r in range(4):
            cout[r].wait()

    return k(data, starts)


def kernel(data, batch_size, length, p):
    del batch_size, length, p
    n = data.shape[0]
    starts = jnp.asarray(_crop_starts(n), dtype=jnp.int32)
    return _crops_sc(data, starts).reshape(_ROWS, _LEN, 1)
